# Initial kernel scaffold; baseline (speedup 1.0000x reference)
import jax
import jax.numpy as jnp
from jax.experimental import pallas as pl


def kernel(x, edge_index, W1, b1, W2, b2):
    raise NotImplementedError("write your pallas kernel here")


# speedup vs baseline: 15.3182x; 15.3182x over previous
import functools

import jax
import jax.numpy as jnp
from jax import lax
from jax.experimental import pallas as pl
from jax.experimental.pallas import tpu as pltpu
from jax.experimental.pallas import tpu_sc as plsc

N = 10000
F = 128
NC, NS = 2, 16
NW = NC * NS
CHUNK = 128
NCH = 80
EPT = NCH * CHUNK
E_PAD = NW * EPT
RPT = 640
ACC_N = NS * RPT
R = 1000
G = N // R

_mesh = plsc.VectorSubcoreMesh(core_axis_name="c", subcore_axis_name="s")


@functools.partial(
    pl.kernel,
    out_type=(jax.ShapeDtypeStruct((ACC_N,), jnp.float32),
              jax.ShapeDtypeStruct((ACC_N,), jnp.float32)),
    mesh=_mesh,
    scratch_types=(
        pltpu.VMEM((NCH, CHUNK), jnp.int32),
        pltpu.VMEM((CHUNK,), jnp.float32),
        pltpu.VMEM((CHUNK,), jnp.float32),
        pltpu.VMEM_SHARED((ACC_N,), jnp.float32),
    ),
)
def _deg_kernel(dst_hbm, d0_hbm, d1_hbm, dst_v, ones_v, zrow_v, acc):
    c = lax.axis_index("c")
    s = lax.axis_index("s")
    for i in range(CHUNK // 16):
        ones_v[pl.ds(i * 16, 16)] = jnp.ones((16,), jnp.float32)
        zrow_v[pl.ds(i * 16, 16)] = jnp.zeros((16,), jnp.float32)
    pltpu.sync_copy(dst_hbm.at[c, s], dst_v)

    def zbody(k, carry):
        pltpu.sync_copy(zrow_v, acc.at[pl.ds(s * RPT + k * CHUNK, CHUNK)])
        return carry
    lax.fori_loop(0, RPT // CHUNK, zbody, 0)
    plsc.subcore_barrier()

    def body(j, carry):
        pltpu.sync_copy(ones_v, acc.at[dst_v.at[j]], add=True)
        return carry
    lax.fori_loop(0, NCH, body, 0)
    plsc.subcore_barrier()

    @pl.when(c == 0)
    def _():
        pltpu.sync_copy(acc.at[pl.ds(s * RPT, RPT)], d0_hbm.at[pl.ds(s * RPT, RPT)])

    @pl.when(c == 1)
    def _():
        pltpu.sync_copy(acc.at[pl.ds(s * RPT, RPT)], d1_hbm.at[pl.ds(s * RPT, RPT)])


@functools.partial(
    pl.kernel,
    out_type=(jax.ShapeDtypeStruct((ACC_N, F), jnp.float32),
              jax.ShapeDtypeStruct((ACC_N, F), jnp.float32)),
    mesh=_mesh,
    scratch_types=(
        pltpu.VMEM((NCH, CHUNK), jnp.int32),
        pltpu.VMEM((NCH, CHUNK), jnp.int32),
        pltpu.VMEM((CHUNK, F), jnp.float32),
        pltpu.VMEM((16, F), jnp.float32),
        pltpu.VMEM_SHARED((ACC_N, F), jnp.float32),
        pltpu.SemaphoreType.DMA,
    ),
)
def _agg_kernel(y_hbm, src_hbm, dst_hbm, p0_hbm, p1_hbm,
                src_v, dst_v, rows_a, zbuf, acc, sem_a):
    c = lax.axis_index("c")
    s = lax.axis_index("s")
    for r in range(16):
        for q in range(F // 16):
            zbuf[r, pl.ds(q * 16, 16)] = jnp.zeros((16,), jnp.float32)
    pltpu.sync_copy(src_hbm.at[c, s], src_v)
    pltpu.sync_copy(dst_hbm.at[c, s], dst_v)

    def zbody(k, carry):
        pltpu.sync_copy(zbuf, acc.at[pl.ds(s * RPT + k * 16, 16)])
        return carry
    lax.fori_loop(0, RPT // 16, zbody, 0)
    plsc.subcore_barrier()

    def body(j, carry):
        pltpu.async_copy(y_hbm.at[src_v.at[j]], rows_a, sem_a).wait()
        pltpu.sync_copy(rows_a, acc.at[dst_v.at[j]], add=True)
        return carry
    lax.fori_loop(0, NCH, body, 0)
    plsc.subcore_barrier()

    @pl.when(c == 0)
    def _():
        pltpu.sync_copy(acc.at[pl.ds(s * RPT, RPT)], p0_hbm.at[pl.ds(s * RPT, RPT)])

    @pl.when(c == 1)
    def _():
        pltpu.sync_copy(acc.at[pl.ds(s * RPT, RPT)], p1_hbm.at[pl.ds(s * RPT, RPT)])


def _prep_body(x_ref, d0_ref, d1_ref, y_ref, dinv_ref):
    deg = d0_ref[...] + d1_ref[...]
    dinv = jnp.where(deg > 0, lax.rsqrt(jnp.maximum(deg, 1.0)), 0.0)
    dinv_ref[...] = dinv
    y_ref[...] = x_ref[...] * dinv


def _prep(x, d0r, d1r):
    return pl.pallas_call(
        _prep_body,
        grid=(G,),
        in_specs=[
            pl.BlockSpec((R, F), lambda i: (i, 0)),
            pl.BlockSpec((R, 1), lambda i: (i, 0)),
            pl.BlockSpec((R, 1), lambda i: (i, 0)),
        ],
        out_specs=[
            pl.BlockSpec((R, F), lambda i: (i, 0)),
            pl.BlockSpec((R, 1), lambda i: (i, 0)),
        ],
        out_shape=[
            jax.ShapeDtypeStruct((N, F), jnp.float32),
            jax.ShapeDtypeStruct((N, 1), jnp.float32),
        ],
    )(x, d0r, d1r)


def _combine_body(h_ref, p0_ref, p1_ref, dinv_ref, w_ref, b_ref,
                  out_ref, y_ref, *, relu, emit_y):
    dinv = dinv_ref[...]
    agg = (p0_ref[...] + p1_ref[...]) * dinv
    acc = jnp.dot(h_ref[...], w_ref[0], preferred_element_type=jnp.float32)
    acc = acc - jnp.dot(agg, w_ref[1], preferred_element_type=jnp.float32)
    acc = acc + b_ref[...]
    if relu:
        acc = jnp.maximum(acc, 0.0)
    out_ref[...] = acc
    if emit_y:
        y_ref[...] = acc * dinv


def _combine(h, p0, p1, dinv, w, b, relu, emit_y):
    body = functools.partial(_combine_body, relu=relu, emit_y=emit_y)
    n_out = 2 if emit_y else 1
    out = pl.pallas_call(
        body if emit_y else (lambda *a: body(*a, None)),
        grid=(G,),
        in_specs=[
            pl.BlockSpec((R, F), lambda i: (i, 0)),
            pl.BlockSpec((R, F), lambda i: (i, 0)),
            pl.BlockSpec((R, F), lambda i: (i, 0)),
            pl.BlockSpec((R, 1), lambda i: (i, 0)),
            pl.BlockSpec((2, F, F), lambda i: (0, 0, 0)),
            pl.BlockSpec((1, F), lambda i: (0, 0)),
        ],
        out_specs=[pl.BlockSpec((R, F), lambda i: (i, 0))] * n_out,
        out_shape=[jax.ShapeDtypeStruct((N, F), jnp.float32)] * n_out,
    )(h, p0, p1, dinv, w, b)
    return out if emit_y else out[0]


def kernel(x, edge_index, W1, b1, W2, b2):
    src = edge_index[0].astype(jnp.int32)
    dst = edge_index[1].astype(jnp.int32)
    npad = E_PAD - src.shape[0]
    ar = jnp.arange(npad, dtype=jnp.int32)
    pad_src = (ar * 131) % N
    pad_dst = N + (ar % (ACC_N - N))
    src_all = jnp.concatenate([src, pad_src]).reshape(NC, NS, NCH, CHUNK)
    dst_all = jnp.concatenate([dst, pad_dst]).reshape(NC, NS, NCH, CHUNK)

    d0, d1 = _deg_kernel(dst_all)
    d0r = d0[:N].reshape(N, 1)
    d1r = d1[:N].reshape(N, 1)
    y, dinv = _prep(x, d0r, d1r)

    b1r = b1.reshape(1, F)
    b2r = b2.reshape(1, F)
    h = x
    for _ in range(2):
        p0, p1 = _agg_kernel(y, src_all, dst_all)
        h, y = _combine(h, p0, p1, dinv, W1, b1r, relu=True, emit_y=True)
    p0, p1 = _agg_kernel(y, src_all, dst_all)
    return _combine(h, p0, p1, dinv, W2, b2r, relu=False, emit_y=False)


# speedup vs baseline: 19.7115x; 1.2868x over previous
import functools

import jax
import jax.numpy as jnp
from jax import lax
from jax.experimental import pallas as pl
from jax.experimental.pallas import tpu as pltpu
from jax.experimental.pallas import tpu_sc as plsc

N = 10000
F = 128
NC, NS = 2, 16
NW = NC * NS
CHUNK = 128
NCH = 80
BCH = 8
NBLK = NCH // BCH
EPT = NCH * CHUNK
E_PAD = NW * EPT
RPT = 640
ACC_N = NS * RPT
R = 1000
G = N // R

_mesh = plsc.VectorSubcoreMesh(core_axis_name="c", subcore_axis_name="s")


@functools.partial(
    pl.kernel,
    out_type=(jax.ShapeDtypeStruct((ACC_N,), jnp.float32),
              jax.ShapeDtypeStruct((ACC_N,), jnp.float32)),
    mesh=_mesh,
    scratch_types=(
        pltpu.VMEM((NBLK, BCH, CHUNK), jnp.int32),
        pltpu.VMEM((CHUNK,), jnp.float32),
        pltpu.VMEM((CHUNK,), jnp.float32),
        pltpu.VMEM_SHARED((ACC_N,), jnp.float32),
    ),
)
def _deg_kernel(dst_hbm, d0_hbm, d1_hbm, dst_v, ones_v, zrow_v, acc):
    c = lax.axis_index("c")
    s = lax.axis_index("s")
    for i in range(CHUNK // 16):
        ones_v[pl.ds(i * 16, 16)] = jnp.ones((16,), jnp.float32)
        zrow_v[pl.ds(i * 16, 16)] = jnp.zeros((16,), jnp.float32)
    pltpu.sync_copy(dst_hbm.at[c, s], dst_v)

    def zbody(k, carry):
        pltpu.sync_copy(zrow_v, acc.at[pl.ds(s * RPT + k * CHUNK, CHUNK)])
        return carry
    lax.fori_loop(0, RPT // CHUNK, zbody, 0)
    plsc.subcore_barrier()

    def body(b, carry):
        for k in range(BCH):
            pltpu.sync_copy(ones_v, acc.at[dst_v.at[b, k]], add=True)
        return carry
    lax.fori_loop(0, NBLK, body, 0)
    plsc.subcore_barrier()

    @pl.when(c == 0)
    def _():
        pltpu.sync_copy(acc.at[pl.ds(s * RPT, RPT)], d0_hbm.at[pl.ds(s * RPT, RPT)])

    @pl.when(c == 1)
    def _():
        pltpu.sync_copy(acc.at[pl.ds(s * RPT, RPT)], d1_hbm.at[pl.ds(s * RPT, RPT)])


@functools.partial(
    pl.kernel,
    out_type=(jax.ShapeDtypeStruct((ACC_N, F), jnp.float32),
              jax.ShapeDtypeStruct((ACC_N, F), jnp.float32)),
    mesh=_mesh,
    scratch_types=(
        pltpu.VMEM((2, BCH, CHUNK), jnp.int32),
        pltpu.VMEM((2, BCH, CHUNK), jnp.int32),
        pltpu.VMEM((CHUNK, F), jnp.float32),
        pltpu.VMEM((CHUNK, F), jnp.float32),
        pltpu.VMEM((16, F), jnp.float32),
        pltpu.VMEM_SHARED((ACC_N, F), jnp.float32),
        pltpu.SemaphoreType.DMA,
        pltpu.SemaphoreType.DMA,
        pltpu.SemaphoreType.DMA,
        pltpu.SemaphoreType.DMA,
    ),
)
def _agg_kernel(y_hbm, src_hbm, dst_hbm, p0_hbm, p1_hbm,
                sidx, didx, rows_a, rows_b, zbuf, acc,
                sem_ga, sem_gb, sem_i0, sem_i1):
    c = lax.axis_index("c")
    s = lax.axis_index("s")
    rows = (rows_a, rows_b)
    sems = (sem_ga, sem_gb)
    for r in range(16):
        for q in range(F // 16):
            zbuf[r, pl.ds(q * 16, 16)] = jnp.zeros((16,), jnp.float32)
    pltpu.sync_copy(src_hbm.at[c, s, 0], sidx.at[0])
    pltpu.sync_copy(dst_hbm.at[c, s, 0], didx.at[0])
    pltpu.async_copy(src_hbm.at[c, s, 1], sidx.at[1], sem_i1)
    pltpu.async_copy(dst_hbm.at[c, s, 1], didx.at[1], sem_i1)

    def zbody(k, carry):
        pltpu.sync_copy(zbuf, acc.at[pl.ds(s * RPT + k * 16, 16)])
        return carry
    lax.fori_loop(0, RPT // 16, zbody, 0)
    plsc.subcore_barrier()

    pltpu.async_copy(y_hbm.at[sidx.at[0, 0]], rows_a, sem_ga)

    def body(bb, carry):
        blk_a = 2 * bb
        for k in range(BCH):
            cur, nxt = rows[k % 2], rows[(k + 1) % 2]
            scur, snxt = sems[k % 2], sems[(k + 1) % 2]
            pltpu.make_async_copy(y_hbm.at[sidx.at[0, k]], cur, scur).wait()
            if k < BCH - 1:
                pltpu.async_copy(y_hbm.at[sidx.at[0, k + 1]], nxt, snxt)
            else:
                pltpu.make_async_copy(src_hbm.at[c, s, 0], sidx.at[1], sem_i1).wait()
                pltpu.make_async_copy(dst_hbm.at[c, s, 0], didx.at[1], sem_i1).wait()
                pltpu.async_copy(y_hbm.at[sidx.at[1, 0]], nxt, snxt)
            pltpu.sync_copy(cur, acc.at[didx.at[0, k]], add=True)

        @pl.when(blk_a + 2 < NBLK)
        def _():
            pltpu.async_copy(src_hbm.at[c, s, blk_a + 2], sidx.at[0], sem_i0)
            pltpu.async_copy(dst_hbm.at[c, s, blk_a + 2], didx.at[0], sem_i0)

        for k in range(BCH):
            cur, nxt = rows[k % 2], rows[(k + 1) % 2]
            scur, snxt = sems[k % 2], sems[(k + 1) % 2]
            pltpu.make_async_copy(y_hbm.at[sidx.at[1, k]], cur, scur).wait()
            if k < BCH - 1:
                pltpu.async_copy(y_hbm.at[sidx.at[1, k + 1]], nxt, snxt)
            else:
                @pl.when(blk_a + 2 < NBLK)
                def _():
                    pltpu.make_async_copy(src_hbm.at[c, s, 0], sidx.at[0], sem_i0).wait()
                    pltpu.make_async_copy(dst_hbm.at[c, s, 0], didx.at[0], sem_i0).wait()
                    pltpu.async_copy(y_hbm.at[sidx.at[0, 0]], nxt, snxt)
            pltpu.sync_copy(cur, acc.at[didx.at[1, k]], add=True)

        @pl.when(blk_a + 3 < NBLK)
        def _():
            pltpu.async_copy(src_hbm.at[c, s, blk_a + 3], sidx.at[1], sem_i1)
            pltpu.async_copy(dst_hbm.at[c, s, blk_a + 3], didx.at[1], sem_i1)
        return carry
    lax.fori_loop(0, NBLK // 2, body, 0)
    plsc.subcore_barrier()

    @pl.when(c == 0)
    def _():
        pltpu.sync_copy(acc.at[pl.ds(s * RPT, RPT)], p0_hbm.at[pl.ds(s * RPT, RPT)])

    @pl.when(c == 1)
    def _():
        pltpu.sync_copy(acc.at[pl.ds(s * RPT, RPT)], p1_hbm.at[pl.ds(s * RPT, RPT)])


def _prep_body(x_ref, d0_ref, d1_ref, y_ref, dinv_ref):
    deg = d0_ref[...] + d1_ref[...]
    dinv = jnp.where(deg > 0, lax.rsqrt(jnp.maximum(deg, 1.0)), 0.0)
    dinv_ref[...] = dinv
    y_ref[...] = x_ref[...] * dinv


def _prep(x, d0r, d1r):
    return pl.pallas_call(
        _prep_body,
        grid=(G,),
        in_specs=[
            pl.BlockSpec((R, F), lambda i: (i, 0)),
            pl.BlockSpec((R, 1), lambda i: (i, 0)),
            pl.BlockSpec((R, 1), lambda i: (i, 0)),
        ],
        out_specs=[
            pl.BlockSpec((R, F), lambda i: (i, 0)),
            pl.BlockSpec((R, 1), lambda i: (i, 0)),
        ],
        out_shape=[
            jax.ShapeDtypeStruct((N, F), jnp.float32),
            jax.ShapeDtypeStruct((N, 1), jnp.float32),
        ],
    )(x, d0r, d1r)


def _combine_body(h_ref, p0_ref, p1_ref, dinv_ref, w_ref, b_ref,
                  out_ref, y_ref, *, relu, emit_y):
    dinv = dinv_ref[...]
    agg = (p0_ref[...] + p1_ref[...]) * dinv
    acc = jnp.dot(h_ref[...], w_ref[0], preferred_element_type=jnp.float32)
    acc = acc - jnp.dot(agg, w_ref[1], preferred_element_type=jnp.float32)
    acc = acc + b_ref[...]
    if relu:
        acc = jnp.maximum(acc, 0.0)
    out_ref[...] = acc
    if emit_y:
        y_ref[...] = acc * dinv


def _combine(h, p0, p1, dinv, w, b, relu, emit_y):
    body = functools.partial(_combine_body, relu=relu, emit_y=emit_y)
    n_out = 2 if emit_y else 1
    out = pl.pallas_call(
        body if emit_y else (lambda *a: body(*a, None)),
        grid=(G,),
        in_specs=[
            pl.BlockSpec((R, F), lambda i: (i, 0)),
            pl.BlockSpec((R, F), lambda i: (i, 0)),
            pl.BlockSpec((R, F), lambda i: (i, 0)),
            pl.BlockSpec((R, 1), lambda i: (i, 0)),
            pl.BlockSpec((2, F, F), lambda i: (0, 0, 0)),
            pl.BlockSpec((1, F), lambda i: (0, 0)),
        ],
        out_specs=[pl.BlockSpec((R, F), lambda i: (i, 0))] * n_out,
        out_shape=[jax.ShapeDtypeStruct((N, F), jnp.float32)] * n_out,
    )(h, p0, p1, dinv, w, b)
    return out if emit_y else out[0]


def kernel(x, edge_index, W1, b1, W2, b2):
    src = edge_index[0].astype(jnp.int32)
    dst = edge_index[1].astype(jnp.int32)
    npad = E_PAD - src.shape[0]
    ar = jnp.arange(npad, dtype=jnp.int32)
    pad_src = (ar * 131) % N
    pad_dst = N + (ar % (ACC_N - N))
    src_all = jnp.concatenate([src, pad_src]).reshape(NC, NS, NBLK, BCH, CHUNK)
    dst_all = jnp.concatenate([dst, pad_dst]).reshape(NC, NS, NBLK, BCH, CHUNK)

    d0, d1 = _deg_kernel(dst_all)
    d0r = d0[:N].reshape(N, 1)
    d1r = d1[:N].reshape(N, 1)
    y, dinv = _prep(x, d0r, d1r)

    b1r = b1.reshape(1, F)
    b2r = b2.reshape(1, F)
    h = x
    for _ in range(2):
        p0, p1 = _agg_kernel(y, src_all, dst_all)
        h, y = _combine(h, p0, p1, dinv, W1, b1r, relu=True, emit_y=True)
    p0, p1 = _agg_kernel(y, src_all, dst_all)
    return _combine(h, p0, p1, dinv, W2, b2r, relu=False, emit_y=False)


# speedup vs baseline: 24.6440x; 1.2502x over previous
import functools

import jax
import jax.numpy as jnp
from jax import lax
from jax.experimental import pallas as pl
from jax.experimental.pallas import tpu as pltpu
from jax.experimental.pallas import tpu_sc as plsc

N = 10000
F = 128
NC, NS = 2, 16
NW = NC * NS
CHUNK = 128
NCH = 81
EPT = NCH * CHUNK
E_PAD = NW * EPT
ACC_N = 10112
RPT = ACC_N // NS
DEG_N = 10240
R = 1000
G = N // R

_mesh = plsc.VectorSubcoreMesh(core_axis_name="c", subcore_axis_name="s")


@functools.partial(
    pl.kernel,
    out_type=(jax.ShapeDtypeStruct((DEG_N,), jnp.float32),
              jax.ShapeDtypeStruct((DEG_N,), jnp.float32)),
    mesh=_mesh,
    scratch_types=(
        pltpu.VMEM((NCH, CHUNK), jnp.int32),
        pltpu.VMEM((CHUNK,), jnp.float32),
        pltpu.VMEM((CHUNK,), jnp.float32),
        pltpu.VMEM_SHARED((DEG_N,), jnp.float32),
    ),
)
def _deg_kernel(dst_hbm, d0_hbm, d1_hbm, dst_v, ones_v, zrow_v, acc):
    c = lax.axis_index("c")
    s = lax.axis_index("s")
    for i in range(CHUNK // 16):
        ones_v[pl.ds(i * 16, 16)] = jnp.ones((16,), jnp.float32)
        zrow_v[pl.ds(i * 16, 16)] = jnp.zeros((16,), jnp.float32)
    pltpu.sync_copy(dst_hbm.at[c, s], dst_v)

    rpt = DEG_N // NS
    def zbody(k, carry):
        pltpu.sync_copy(zrow_v, acc.at[pl.ds(s * rpt + k * CHUNK, CHUNK)])
        return carry
    lax.fori_loop(0, rpt // CHUNK, zbody, 0)
    plsc.subcore_barrier()

    def body(j, carry):
        pltpu.sync_copy(ones_v, acc.at[dst_v.at[j]], add=True)
        return carry
    lax.fori_loop(0, NCH, body, 0)
    plsc.subcore_barrier()

    @pl.when(c == 0)
    def _():
        pltpu.sync_copy(acc.at[pl.ds(s * rpt, rpt)], d0_hbm.at[pl.ds(s * rpt, rpt)])

    @pl.when(c == 1)
    def _():
        pltpu.sync_copy(acc.at[pl.ds(s * rpt, rpt)], d1_hbm.at[pl.ds(s * rpt, rpt)])


@functools.partial(
    pl.kernel,
    out_type=(jax.ShapeDtypeStruct((ACC_N, F), jnp.float32),
              jax.ShapeDtypeStruct((ACC_N, F), jnp.float32)),
    mesh=_mesh,
    scratch_types=(
        pltpu.VMEM((3, CHUNK), jnp.int32),
        pltpu.VMEM((3, CHUNK), jnp.int32),
        pltpu.VMEM((CHUNK, F), jnp.float32),
        pltpu.VMEM((CHUNK, F), jnp.float32),
        pltpu.VMEM((CHUNK, F), jnp.float32),
        pltpu.VMEM_SHARED((ACC_N, F), jnp.float32),
        pltpu.SemaphoreType.DMA,
        pltpu.SemaphoreType.DMA,
        pltpu.SemaphoreType.DMA,
        pltpu.SemaphoreType.DMA,
        pltpu.SemaphoreType.DMA,
        pltpu.SemaphoreType.DMA,
        pltpu.SemaphoreType.DMA,
        pltpu.SemaphoreType.DMA,
        pltpu.SemaphoreType.DMA,
        pltpu.SemaphoreType.DMA,
        pltpu.SemaphoreType.DMA,
        pltpu.SemaphoreType.DMA,
    ),
)
def _agg_kernel(y_hbm, src_hbm, dst_hbm, p0_hbm, p1_hbm,
                sidx, didx, rb0, rb1, rb2, acc,
                sg0, sg1, sg2, ss0, ss1, ss2,
                si0, si1, si2, sd0, sd1, sd2):
    c = lax.axis_index("c")
    s = lax.axis_index("s")
    rows = (rb0, rb1, rb2)
    sg = (sg0, sg1, sg2)
    ss = (ss0, ss1, ss2)
    si = (si0, si1, si2)
    sd = (sd0, sd1, sd2)

    def issue_gather(slot, buf_i):
        pltpu.async_copy(y_hbm.at[sidx.at[slot]], rows[buf_i], sg[buf_i])

    def wait_gather(buf_i):
        pltpu.make_async_copy(y_hbm.at[sidx.at[0]], rows[buf_i], sg[buf_i]).wait()

    def issue_scatter(slot, buf_i):
        pltpu.async_copy(rows[buf_i], acc.at[didx.at[slot]], ss[buf_i], add=True)

    def wait_scatter(buf_i):
        pltpu.make_async_copy(rows[buf_i], acc.at[didx.at[0]], ss[buf_i]).wait()

    for r in range(16):
        for q in range(F // 16):
            rb2[r, pl.ds(q * 16, 16)] = jnp.zeros((16,), jnp.float32)

    def zbody(k, carry):
        pltpu.sync_copy(rb2.at[pl.ds(0, 16)], acc.at[pl.ds(s * RPT + k * 16, 16)])
        return carry
    lax.fori_loop(0, RPT // 16, zbody, 0)
    pltpu.sync_copy(rb2.at[pl.ds(0, RPT % 16)],
                    acc.at[pl.ds(s * RPT + (RPT // 16) * 16, RPT % 16)])

    for j in range(3):
        pltpu.sync_copy(src_hbm.at[c, s, j], sidx.at[j])
        pltpu.sync_copy(dst_hbm.at[c, s, j], didx.at[j])
    plsc.subcore_barrier()

    issue_gather(0, 0)
    issue_gather(1, 1)

    def body(t, carry):
        for u in range(3):
            j = 3 * t + u
            u1 = (u + 1) % 3
            u2 = (u + 2) % 3

            @pl.when(j >= 1)
            def _():
                wait_scatter(u2)

            @pl.when(jnp.logical_and(j >= 1, j + 2 < NCH))
            def _():
                pltpu.async_copy(dst_hbm.at[c, s, j + 2], didx.at[u2], sd[u2])

            @pl.when(jnp.logical_and(j >= 1, j + 2 < NCH))
            def _():
                pltpu.make_async_copy(
                    src_hbm.at[c, s, 0], sidx.at[u2], si[u2]).wait()

            @pl.when(j + 2 < NCH)
            def _():
                issue_gather(u2, u2)

            wait_gather(u)

            @pl.when(j + 3 < NCH)
            def _():
                pltpu.async_copy(src_hbm.at[c, s, j + 3], sidx.at[u], si[u])

            @pl.when(j >= 3)
            def _():
                pltpu.make_async_copy(
                    dst_hbm.at[c, s, 0], didx.at[u], sd[u]).wait()

            issue_scatter(u, u)
        return carry
    lax.fori_loop(0, NCH // 3, body, 0)
    wait_scatter((NCH - 1) % 3)
    plsc.subcore_barrier()

    @pl.when(c == 0)
    def _():
        pltpu.sync_copy(acc.at[pl.ds(s * RPT, RPT)], p0_hbm.at[pl.ds(s * RPT, RPT)])

    @pl.when(c == 1)
    def _():
        pltpu.sync_copy(acc.at[pl.ds(s * RPT, RPT)], p1_hbm.at[pl.ds(s * RPT, RPT)])


def _prep_body(x_ref, d0_ref, d1_ref, y_ref, dinv_ref):
    deg = d0_ref[...] + d1_ref[...]
    dinv = jnp.where(deg > 0, lax.rsqrt(jnp.maximum(deg, 1.0)), 0.0)
    dinv_ref[...] = dinv
    y_ref[...] = x_ref[...] * dinv


def _prep(x, d0r, d1r):
    return pl.pallas_call(
        _prep_body,
        grid=(G,),
        in_specs=[
            pl.BlockSpec((R, F), lambda i: (i, 0)),
            pl.BlockSpec((R, 1), lambda i: (i, 0)),
            pl.BlockSpec((R, 1), lambda i: (i, 0)),
        ],
        out_specs=[
            pl.BlockSpec((R, F), lambda i: (i, 0)),
            pl.BlockSpec((R, 1), lambda i: (i, 0)),
        ],
        out_shape=[
            jax.ShapeDtypeStruct((N, F), jnp.float32),
            jax.ShapeDtypeStruct((N, 1), jnp.float32),
        ],
    )(x, d0r, d1r)


def _combine_body(h_ref, p0_ref, p1_ref, dinv_ref, w_ref, b_ref,
                  out_ref, y_ref, *, relu, emit_y):
    dinv = dinv_ref[...]
    agg = (p0_ref[...] + p1_ref[...]) * dinv
    acc = jnp.dot(h_ref[...], w_ref[0], preferred_element_type=jnp.float32)
    acc = acc - jnp.dot(agg, w_ref[1], preferred_element_type=jnp.float32)
    acc = acc + b_ref[...]
    if relu:
        acc = jnp.maximum(acc, 0.0)
    out_ref[...] = acc
    if emit_y:
        y_ref[...] = acc * dinv


def _combine(h, p0, p1, dinv, w, b, relu, emit_y):
    body = functools.partial(_combine_body, relu=relu, emit_y=emit_y)
    n_out = 2 if emit_y else 1
    out = pl.pallas_call(
        body if emit_y else (lambda *a: body(*a, None)),
        grid=(G,),
        in_specs=[
            pl.BlockSpec((R, F), lambda i: (i, 0)),
            pl.BlockSpec((R, F), lambda i: (i, 0)),
            pl.BlockSpec((R, F), lambda i: (i, 0)),
            pl.BlockSpec((R, 1), lambda i: (i, 0)),
            pl.BlockSpec((2, F, F), lambda i: (0, 0, 0)),
            pl.BlockSpec((1, F), lambda i: (0, 0)),
        ],
        out_specs=[pl.BlockSpec((R, F), lambda i: (i, 0))] * n_out,
        out_shape=[jax.ShapeDtypeStruct((N, F), jnp.float32)] * n_out,
    )(h, p0, p1, dinv, w, b)
    return out if emit_y else out[0]


def kernel(x, edge_index, W1, b1, W2, b2):
    src = edge_index[0].astype(jnp.int32)
    dst = edge_index[1].astype(jnp.int32)
    npad = E_PAD - src.shape[0]
    ar = jnp.arange(npad, dtype=jnp.int32)
    pad_src = (ar * 131) % N
    pad_dst = N + (ar % (ACC_N - N))
    src_all = jnp.concatenate([src, pad_src]).reshape(NC, NS, NCH, CHUNK)
    dst_all = jnp.concatenate([dst, pad_dst]).reshape(NC, NS, NCH, CHUNK)

    d0, d1 = _deg_kernel(dst_all)
    d0r = d0[:N].reshape(N, 1)
    d1r = d1[:N].reshape(N, 1)
    y, dinv = _prep(x, d0r, d1r)

    b1r = b1.reshape(1, F)
    b2r = b2.reshape(1, F)
    h = x
    for _ in range(2):
        p0, p1 = _agg_kernel(y, src_all, dst_all)
        h, y = _combine(h, p0, p1, dinv, W1, b1r, relu=True, emit_y=True)
    p0, p1 = _agg_kernel(y, src_all, dst_all)
    return _combine(h, p0, p1, dinv, W2, b2r, relu=False, emit_y=False)


# speedup vs baseline: 25.2322x; 1.0239x over previous
import functools

import jax
import jax.numpy as jnp
from jax import lax
from jax.experimental import pallas as pl
from jax.experimental.pallas import tpu as pltpu
from jax.experimental.pallas import tpu_sc as plsc

N = 10000
F = 128
NC, NS = 2, 16
NW = NC * NS
CHUNK = 128
NCH = 81
EPT = NCH * CHUNK
E_PAD = NW * EPT
ACC_N = 10112
RPT = ACC_N // NS
DEG_N = 10240
R = 1000
G = N // R

_mesh = plsc.VectorSubcoreMesh(core_axis_name="c", subcore_axis_name="s")


@functools.partial(
    pl.kernel,
    out_type=(jax.ShapeDtypeStruct((DEG_N,), jnp.float32),
              jax.ShapeDtypeStruct((DEG_N,), jnp.float32)),
    mesh=_mesh,
    scratch_types=(
        pltpu.VMEM((NCH, CHUNK), jnp.int32),
        pltpu.VMEM((CHUNK,), jnp.float32),
        pltpu.VMEM((CHUNK,), jnp.float32),
        pltpu.VMEM_SHARED((DEG_N,), jnp.float32),
    ),
)
def _deg_kernel(dst_hbm, d0_hbm, d1_hbm, dst_v, ones_v, zrow_v, acc):
    c = lax.axis_index("c")
    s = lax.axis_index("s")
    for i in range(CHUNK // 16):
        ones_v[pl.ds(i * 16, 16)] = jnp.ones((16,), jnp.float32)
        zrow_v[pl.ds(i * 16, 16)] = jnp.zeros((16,), jnp.float32)
    pltpu.sync_copy(dst_hbm.at[c, s], dst_v)

    rpt = DEG_N // NS
    def zbody(k, carry):
        pltpu.sync_copy(zrow_v, acc.at[pl.ds(s * rpt + k * CHUNK, CHUNK)])
        return carry
    lax.fori_loop(0, rpt // CHUNK, zbody, 0)
    plsc.subcore_barrier()

    def body(j, carry):
        pltpu.sync_copy(ones_v, acc.at[dst_v.at[j]], add=True)
        return carry
    lax.fori_loop(0, NCH, body, 0)
    plsc.subcore_barrier()

    @pl.when(c == 0)
    def _():
        pltpu.sync_copy(acc.at[pl.ds(s * rpt, rpt)], d0_hbm.at[pl.ds(s * rpt, rpt)])

    @pl.when(c == 1)
    def _():
        pltpu.sync_copy(acc.at[pl.ds(s * rpt, rpt)], d1_hbm.at[pl.ds(s * rpt, rpt)])


@functools.partial(
    pl.kernel,
    out_type=(jax.ShapeDtypeStruct((ACC_N, F), jnp.float32),
              jax.ShapeDtypeStruct((ACC_N, F), jnp.float32)),
    mesh=_mesh,
    scratch_types=(
        pltpu.VMEM((3, CHUNK), jnp.int32),
        pltpu.VMEM((3, CHUNK), jnp.int32),
        pltpu.VMEM((CHUNK, F), jnp.float32),
        pltpu.VMEM((CHUNK, F), jnp.float32),
        pltpu.VMEM((CHUNK, F), jnp.float32),
        pltpu.VMEM_SHARED((ACC_N, F), jnp.float32),
        pltpu.SemaphoreType.DMA,
        pltpu.SemaphoreType.DMA,
        pltpu.SemaphoreType.DMA,
        pltpu.SemaphoreType.DMA,
        pltpu.SemaphoreType.DMA,
        pltpu.SemaphoreType.DMA,
        pltpu.SemaphoreType.DMA,
        pltpu.SemaphoreType.DMA,
        pltpu.SemaphoreType.DMA,
        pltpu.SemaphoreType.DMA,
        pltpu.SemaphoreType.DMA,
        pltpu.SemaphoreType.DMA,
    ),
)
def _agg_kernel(y_hbm, src_hbm, dst_hbm, p0_hbm, p1_hbm,
                sidx, didx, rb0, rb1, rb2, acc,
                sg0, sg1, sg2, ss0, ss1, ss2,
                si0, si1, si2, sd0, sd1, sd2):
    c = lax.axis_index("c")
    s = lax.axis_index("s")
    rows = (rb0, rb1, rb2)
    sg = (sg0, sg1, sg2)
    ss = (ss0, ss1, ss2)
    si = (si0, si1, si2)
    sd = (sd0, sd1, sd2)

    def issue_gather(slot, buf_i):
        pltpu.async_copy(y_hbm.at[sidx.at[slot]], rows[buf_i], sg[buf_i])

    def wait_gather(buf_i):
        pltpu.make_async_copy(y_hbm.at[sidx.at[0]], rows[buf_i], sg[buf_i]).wait()

    def issue_scatter(slot, buf_i):
        pltpu.async_copy(rows[buf_i], acc.at[didx.at[slot]], ss[buf_i], add=True)

    def wait_scatter(buf_i):
        pltpu.make_async_copy(rows[buf_i], acc.at[didx.at[0]], ss[buf_i]).wait()

    for r in range(16):
        for q in range(F // 16):
            rb2[r, pl.ds(q * 16, 16)] = jnp.zeros((16,), jnp.float32)

    def zbody(k, carry):
        pltpu.async_copy(rb2.at[pl.ds(0, 16)],
                         acc.at[pl.ds(s * RPT + k * 16, 16)], sg0)
        return carry
    lax.fori_loop(0, RPT // 16, zbody, 0)
    pltpu.async_copy(rb2.at[pl.ds(0, RPT % 16)],
                     acc.at[pl.ds(s * RPT + (RPT // 16) * 16, RPT % 16)], sg0)

    for j in range(3):
        pltpu.sync_copy(src_hbm.at[c, s, j], sidx.at[j])
        pltpu.sync_copy(dst_hbm.at[c, s, j], didx.at[j])

    def zdrain(k, carry):
        pltpu.make_async_copy(rb2.at[pl.ds(0, 16)],
                              acc.at[pl.ds(s * RPT + k * 16, 16)], sg0).wait()
        return carry
    lax.fori_loop(0, RPT // 16, zdrain, 0)
    pltpu.make_async_copy(rb2.at[pl.ds(0, RPT % 16)],
                          acc.at[pl.ds(s * RPT + (RPT // 16) * 16, RPT % 16)],
                          sg0).wait()
    plsc.subcore_barrier()

    issue_gather(0, 0)
    issue_gather(1, 1)

    def body(t, carry):
        for u in range(3):
            j = 3 * t + u
            u1 = (u + 1) % 3
            u2 = (u + 2) % 3

            @pl.when(j >= 1)
            def _():
                wait_scatter(u2)

            @pl.when(jnp.logical_and(j >= 1, j + 2 < NCH))
            def _():
                pltpu.async_copy(dst_hbm.at[c, s, j + 2], didx.at[u2], sd[u2])

            @pl.when(jnp.logical_and(j >= 1, j + 2 < NCH))
            def _():
                pltpu.make_async_copy(
                    src_hbm.at[c, s, 0], sidx.at[u2], si[u2]).wait()

            @pl.when(j + 2 < NCH)
            def _():
                issue_gather(u2, u2)

            wait_gather(u)

            @pl.when(j + 3 < NCH)
            def _():
                pltpu.async_copy(src_hbm.at[c, s, j + 3], sidx.at[u], si[u])

            @pl.when(j >= 3)
            def _():
                pltpu.make_async_copy(
                    dst_hbm.at[c, s, 0], didx.at[u], sd[u]).wait()

            issue_scatter(u, u)
        return carry
    lax.fori_loop(0, NCH // 3, body, 0)
    wait_scatter((NCH - 1) % 3)
    plsc.subcore_barrier()

    @pl.when(c == 0)
    def _():
        pltpu.sync_copy(acc.at[pl.ds(s * RPT, RPT)], p0_hbm.at[pl.ds(s * RPT, RPT)])

    @pl.when(c == 1)
    def _():
        pltpu.sync_copy(acc.at[pl.ds(s * RPT, RPT)], p1_hbm.at[pl.ds(s * RPT, RPT)])


def _prep_body(x_ref, d0_ref, d1_ref, y_ref, dinv_ref):
    deg = d0_ref[...] + d1_ref[...]
    dinv = jnp.where(deg > 0, lax.rsqrt(jnp.maximum(deg, 1.0)), 0.0)
    dinv_ref[...] = dinv
    y_ref[...] = x_ref[...] * dinv


def _prep(x, d0r, d1r):
    return pl.pallas_call(
        _prep_body,
        grid=(G,),
        in_specs=[
            pl.BlockSpec((R, F), lambda i: (i, 0)),
            pl.BlockSpec((R, 1), lambda i: (i, 0)),
            pl.BlockSpec((R, 1), lambda i: (i, 0)),
        ],
        out_specs=[
            pl.BlockSpec((R, F), lambda i: (i, 0)),
            pl.BlockSpec((R, 1), lambda i: (i, 0)),
        ],
        out_shape=[
            jax.ShapeDtypeStruct((N, F), jnp.float32),
            jax.ShapeDtypeStruct((N, 1), jnp.float32),
        ],
    )(x, d0r, d1r)


def _combine_body(h_ref, p0_ref, p1_ref, dinv_ref, w_ref, b_ref,
                  out_ref, y_ref, *, relu, emit_y):
    dinv = dinv_ref[...]
    agg = (p0_ref[...] + p1_ref[...]) * dinv
    acc = jnp.dot(h_ref[...], w_ref[0], preferred_element_type=jnp.float32)
    acc = acc - jnp.dot(agg, w_ref[1], preferred_element_type=jnp.float32)
    acc = acc + b_ref[...]
    if relu:
        acc = jnp.maximum(acc, 0.0)
    out_ref[...] = acc
    if emit_y:
        y_ref[...] = acc * dinv


def _combine(h, p0, p1, dinv, w, b, relu, emit_y):
    body = functools.partial(_combine_body, relu=relu, emit_y=emit_y)
    n_out = 2 if emit_y else 1
    out = pl.pallas_call(
        body if emit_y else (lambda *a: body(*a, None)),
        grid=(G,),
        in_specs=[
            pl.BlockSpec((R, F), lambda i: (i, 0)),
            pl.BlockSpec((R, F), lambda i: (i, 0)),
            pl.BlockSpec((R, F), lambda i: (i, 0)),
            pl.BlockSpec((R, 1), lambda i: (i, 0)),
            pl.BlockSpec((2, F, F), lambda i: (0, 0, 0)),
            pl.BlockSpec((1, F), lambda i: (0, 0)),
        ],
        out_specs=[pl.BlockSpec((R, F), lambda i: (i, 0))] * n_out,
        out_shape=[jax.ShapeDtypeStruct((N, F), jnp.float32)] * n_out,
    )(h, p0, p1, dinv, w, b)
    return out if emit_y else out[0]


def kernel(x, edge_index, W1, b1, W2, b2):
    src = edge_index[0].astype(jnp.int32)
    dst = edge_index[1].astype(jnp.int32)
    npad = E_PAD - src.shape[0]
    ar = jnp.arange(npad, dtype=jnp.int32)
    pad_src = (ar * 131) % N
    pad_dst = N + (ar % (ACC_N - N))
    src_all = jnp.concatenate([src, pad_src]).reshape(NC, NS, NCH, CHUNK)
    dst_all = jnp.concatenate([dst, pad_dst]).reshape(NC, NS, NCH, CHUNK)

    d0, d1 = _deg_kernel(dst_all)
    d0r = d0[:N].reshape(N, 1)
    d1r = d1[:N].reshape(N, 1)
    y, dinv = _prep(x, d0r, d1r)

    b1r = b1.reshape(1, F)
    b2r = b2.reshape(1, F)
    h = x
    for _ in range(2):
        p0, p1 = _agg_kernel(y, src_all, dst_all)
        h, y = _combine(h, p0, p1, dinv, W1, b1r, relu=True, emit_y=True)
    p0, p1 = _agg_kernel(y, src_all, dst_all)
    return _combine(h, p0, p1, dinv, W2, b2r, relu=False, emit_y=False)


# speedup vs baseline: 25.5802x; 1.0138x over previous
import functools

import jax
import jax.numpy as jnp
from jax import lax
from jax.experimental import pallas as pl
from jax.experimental.pallas import tpu as pltpu
from jax.experimental.pallas import tpu_sc as plsc

N = 10000
F = 128
NC, NS = 2, 16
NW = NC * NS
CHUNK = 128
NCH = 81
EPT = NCH * CHUNK
E_PAD = NW * EPT
ACC_N = 10112
RPT = ACC_N // NS
DEG_N = 10240
R = 1000
G = N // R

_mesh = plsc.VectorSubcoreMesh(core_axis_name="c", subcore_axis_name="s")


@functools.partial(
    pl.kernel,
    out_type=(jax.ShapeDtypeStruct((DEG_N,), jnp.float32),
              jax.ShapeDtypeStruct((DEG_N,), jnp.float32)),
    mesh=_mesh,
    scratch_types=(
        pltpu.VMEM((NCH, CHUNK), jnp.int32),
        pltpu.VMEM((CHUNK,), jnp.float32),
        pltpu.VMEM((CHUNK,), jnp.float32),
        pltpu.VMEM_SHARED((DEG_N,), jnp.float32),
        pltpu.SemaphoreType.DMA,
    ),
)
def _deg_kernel(dst_hbm, d0_hbm, d1_hbm, dst_v, ones_v, zrow_v, acc, sem):
    c = lax.axis_index("c")
    s = lax.axis_index("s")
    for i in range(CHUNK // 16):
        ones_v[pl.ds(i * 16, 16)] = jnp.ones((16,), jnp.float32)
        zrow_v[pl.ds(i * 16, 16)] = jnp.zeros((16,), jnp.float32)
    pltpu.sync_copy(dst_hbm.at[c, s], dst_v)

    rpt = DEG_N // NS
    def zbody(k, carry):
        pltpu.sync_copy(zrow_v, acc.at[pl.ds(s * rpt + k * CHUNK, CHUNK)])
        return carry
    lax.fori_loop(0, rpt // CHUNK, zbody, 0)
    plsc.subcore_barrier()

    def body(j, carry):
        pltpu.async_copy(ones_v, acc.at[dst_v.at[j]], sem, add=True)
        return carry
    lax.fori_loop(0, NCH, body, 0)

    def drain(j, carry):
        pltpu.make_async_copy(ones_v, acc.at[dst_v.at[j]], sem).wait()
        return carry
    lax.fori_loop(0, NCH, drain, 0)
    plsc.subcore_barrier()

    @pl.when(c == 0)
    def _():
        pltpu.sync_copy(acc.at[pl.ds(s * rpt, rpt)], d0_hbm.at[pl.ds(s * rpt, rpt)])

    @pl.when(c == 1)
    def _():
        pltpu.sync_copy(acc.at[pl.ds(s * rpt, rpt)], d1_hbm.at[pl.ds(s * rpt, rpt)])


@functools.partial(
    pl.kernel,
    out_type=(jax.ShapeDtypeStruct((ACC_N, F), jnp.float32),
              jax.ShapeDtypeStruct((ACC_N, F), jnp.float32)),
    mesh=_mesh,
    scratch_types=(
        pltpu.VMEM((3, CHUNK), jnp.int32),
        pltpu.VMEM((3, CHUNK), jnp.int32),
        pltpu.VMEM((CHUNK, F), jnp.float32),
        pltpu.VMEM((CHUNK, F), jnp.float32),
        pltpu.VMEM((CHUNK, F), jnp.float32),
        pltpu.VMEM_SHARED((ACC_N, F), jnp.float32),
        pltpu.SemaphoreType.DMA,
        pltpu.SemaphoreType.DMA,
        pltpu.SemaphoreType.DMA,
        pltpu.SemaphoreType.DMA,
        pltpu.SemaphoreType.DMA,
        pltpu.SemaphoreType.DMA,
        pltpu.SemaphoreType.DMA,
        pltpu.SemaphoreType.DMA,
        pltpu.SemaphoreType.DMA,
        pltpu.SemaphoreType.DMA,
        pltpu.SemaphoreType.DMA,
        pltpu.SemaphoreType.DMA,
    ),
)
def _agg_kernel(y_hbm, src_hbm, dst_hbm, p0_hbm, p1_hbm,
                sidx, didx, rb0, rb1, rb2, acc,
                sg0, sg1, sg2, ss0, ss1, ss2,
                si0, si1, si2, sd0, sd1, sd2):
    c = lax.axis_index("c")
    s = lax.axis_index("s")
    rows = (rb0, rb1, rb2)
    sg = (sg0, sg1, sg2)
    ss = (ss0, ss1, ss2)
    si = (si0, si1, si2)
    sd = (sd0, sd1, sd2)

    def issue_gather(slot, buf_i):
        pltpu.async_copy(y_hbm.at[sidx.at[slot]], rows[buf_i], sg[buf_i])

    def wait_gather(buf_i):
        pltpu.make_async_copy(y_hbm.at[sidx.at[0]], rows[buf_i], sg[buf_i]).wait()

    def issue_scatter(slot, buf_i):
        pltpu.async_copy(rows[buf_i], acc.at[didx.at[slot]], ss[buf_i], add=True)

    def wait_scatter(buf_i):
        pltpu.make_async_copy(rows[buf_i], acc.at[didx.at[0]], ss[buf_i]).wait()

    for r in range(16):
        for q in range(F // 16):
            rb2[r, pl.ds(q * 16, 16)] = jnp.zeros((16,), jnp.float32)

    def zbody(k, carry):
        pltpu.async_copy(rb2.at[pl.ds(0, 16)],
                         acc.at[pl.ds(s * RPT + k * 16, 16)], sg0)
        return carry
    lax.fori_loop(0, RPT // 16, zbody, 0)
    pltpu.async_copy(rb2.at[pl.ds(0, RPT % 16)],
                     acc.at[pl.ds(s * RPT + (RPT // 16) * 16, RPT % 16)], sg0)

    for j in range(3):
        pltpu.sync_copy(src_hbm.at[c, s, j], sidx.at[j])
        pltpu.sync_copy(dst_hbm.at[c, s, j], didx.at[j])

    def zdrain(k, carry):
        pltpu.make_async_copy(rb2.at[pl.ds(0, 16)],
                              acc.at[pl.ds(s * RPT + k * 16, 16)], sg0).wait()
        return carry
    lax.fori_loop(0, RPT // 16, zdrain, 0)
    pltpu.make_async_copy(rb2.at[pl.ds(0, RPT % 16)],
                          acc.at[pl.ds(s * RPT + (RPT // 16) * 16, RPT % 16)],
                          sg0).wait()
    plsc.subcore_barrier()

    issue_gather(0, 0)
    issue_gather(1, 1)

    def body(t, carry):
        for u in range(3):
            j = 3 * t + u
            u1 = (u + 1) % 3
            u2 = (u + 2) % 3

            @pl.when(j >= 1)
            def _():
                wait_scatter(u2)

            @pl.when(jnp.logical_and(j >= 1, j + 2 < NCH))
            def _():
                pltpu.async_copy(dst_hbm.at[c, s, j + 2], didx.at[u2], sd[u2])

            @pl.when(jnp.logical_and(j >= 1, j + 2 < NCH))
            def _():
                pltpu.make_async_copy(
                    src_hbm.at[c, s, 0], sidx.at[u2], si[u2]).wait()

            @pl.when(j + 2 < NCH)
            def _():
                issue_gather(u2, u2)

            wait_gather(u)

            @pl.when(j + 3 < NCH)
            def _():
                pltpu.async_copy(src_hbm.at[c, s, j + 3], sidx.at[u], si[u])

            @pl.when(j >= 3)
            def _():
                pltpu.make_async_copy(
                    dst_hbm.at[c, s, 0], didx.at[u], sd[u]).wait()

            issue_scatter(u, u)
        return carry
    lax.fori_loop(0, NCH // 3, body, 0)
    wait_scatter((NCH - 1) % 3)
    plsc.subcore_barrier()

    @pl.when(c == 0)
    def _():
        pltpu.sync_copy(acc.at[pl.ds(s * RPT, RPT)], p0_hbm.at[pl.ds(s * RPT, RPT)])

    @pl.when(c == 1)
    def _():
        pltpu.sync_copy(acc.at[pl.ds(s * RPT, RPT)], p1_hbm.at[pl.ds(s * RPT, RPT)])


def _prep_body(x_ref, d0_ref, d1_ref, y_ref, dinv_ref):
    deg = d0_ref[...] + d1_ref[...]
    dinv = jnp.where(deg > 0, lax.rsqrt(jnp.maximum(deg, 1.0)), 0.0)
    dinv_ref[...] = dinv
    y_ref[...] = x_ref[...] * dinv


def _prep(x, d0r, d1r):
    return pl.pallas_call(
        _prep_body,
        grid=(G,),
        in_specs=[
            pl.BlockSpec((R, F), lambda i: (i, 0)),
            pl.BlockSpec((R, 1), lambda i: (i, 0)),
            pl.BlockSpec((R, 1), lambda i: (i, 0)),
        ],
        out_specs=[
            pl.BlockSpec((R, F), lambda i: (i, 0)),
            pl.BlockSpec((R, 1), lambda i: (i, 0)),
        ],
        out_shape=[
            jax.ShapeDtypeStruct((N, F), jnp.float32),
            jax.ShapeDtypeStruct((N, 1), jnp.float32),
        ],
    )(x, d0r, d1r)


def _combine_body(h_ref, p0_ref, p1_ref, dinv_ref, w_ref, b_ref,
                  out_ref, y_ref, *, relu, emit_y):
    dinv = dinv_ref[...]
    agg = (p0_ref[...] + p1_ref[...]) * dinv
    acc = jnp.dot(h_ref[...], w_ref[0], preferred_element_type=jnp.float32)
    acc = acc - jnp.dot(agg, w_ref[1], preferred_element_type=jnp.float32)
    acc = acc + b_ref[...]
    if relu:
        acc = jnp.maximum(acc, 0.0)
    out_ref[...] = acc
    if emit_y:
        y_ref[...] = acc * dinv


def _combine(h, p0, p1, dinv, w, b, relu, emit_y):
    body = functools.partial(_combine_body, relu=relu, emit_y=emit_y)
    n_out = 2 if emit_y else 1
    out = pl.pallas_call(
        body if emit_y else (lambda *a: body(*a, None)),
        grid=(G,),
        in_specs=[
            pl.BlockSpec((R, F), lambda i: (i, 0)),
            pl.BlockSpec((R, F), lambda i: (i, 0)),
            pl.BlockSpec((R, F), lambda i: (i, 0)),
            pl.BlockSpec((R, 1), lambda i: (i, 0)),
            pl.BlockSpec((2, F, F), lambda i: (0, 0, 0)),
            pl.BlockSpec((1, F), lambda i: (0, 0)),
        ],
        out_specs=[pl.BlockSpec((R, F), lambda i: (i, 0))] * n_out,
        out_shape=[jax.ShapeDtypeStruct((N, F), jnp.float32)] * n_out,
    )(h, p0, p1, dinv, w, b)
    return out if emit_y else out[0]


def kernel(x, edge_index, W1, b1, W2, b2):
    src = edge_index[0].astype(jnp.int32)
    dst = edge_index[1].astype(jnp.int32)
    npad = E_PAD - src.shape[0]
    ar = jnp.arange(npad, dtype=jnp.int32)
    pad_src = (ar * 131) % N
    pad_dst = N + (ar % (ACC_N - N))
    src_all = jnp.concatenate([src, pad_src]).reshape(NC, NS, NCH, CHUNK)
    dst_all = jnp.concatenate([dst, pad_dst]).reshape(NC, NS, NCH, CHUNK)

    d0, d1 = _deg_kernel(dst_all)
    d0r = d0[:N].reshape(N, 1)
    d1r = d1[:N].reshape(N, 1)
    y, dinv = _prep(x, d0r, d1r)

    b1r = b1.reshape(1, F)
    b2r = b2.reshape(1, F)
    h = x
    for _ in range(2):
        p0, p1 = _agg_kernel(y, src_all, dst_all)
        h, y = _combine(h, p0, p1, dinv, W1, b1r, relu=True, emit_y=True)
    p0, p1 = _agg_kernel(y, src_all, dst_all)
    return _combine(h, p0, p1, dinv, W2, b2r, relu=False, emit_y=False)


# speedup vs baseline: 26.0555x; 1.0186x over previous
import functools

import jax
import jax.numpy as jnp
from jax import lax
from jax.experimental import pallas as pl
from jax.experimental.pallas import tpu as pltpu
from jax.experimental.pallas import tpu_sc as plsc

N = 10000
F = 128
NC, NS = 2, 16
NW = NC * NS
CHUNK = 128
NCH = 79
EPT = NCH * CHUNK
E_PAD = NW * EPT
ACC_N = 10112
RPT = ACC_N // NS
DEG_N = 10240
R = 1000
G = N // R

_mesh = plsc.VectorSubcoreMesh(core_axis_name="c", subcore_axis_name="s")


@functools.partial(
    pl.kernel,
    out_type=(jax.ShapeDtypeStruct((DEG_N,), jnp.float32),
              jax.ShapeDtypeStruct((DEG_N,), jnp.float32)),
    mesh=_mesh,
    scratch_types=(
        pltpu.VMEM((NCH, CHUNK), jnp.int32),
        pltpu.VMEM((CHUNK,), jnp.float32),
        pltpu.VMEM((CHUNK,), jnp.float32),
        pltpu.VMEM_SHARED((DEG_N,), jnp.float32),
        pltpu.SemaphoreType.DMA,
    ),
)
def _deg_kernel(dst_hbm, d0_hbm, d1_hbm, dst_v, ones_v, zrow_v, acc, sem):
    c = lax.axis_index("c")
    s = lax.axis_index("s")
    for i in range(CHUNK // 16):
        ones_v[pl.ds(i * 16, 16)] = jnp.ones((16,), jnp.float32)
        zrow_v[pl.ds(i * 16, 16)] = jnp.zeros((16,), jnp.float32)
    pltpu.sync_copy(dst_hbm.at[c, s], dst_v)

    rpt = DEG_N // NS
    def zbody(k, carry):
        pltpu.sync_copy(zrow_v, acc.at[pl.ds(s * rpt + k * CHUNK, CHUNK)])
        return carry
    lax.fori_loop(0, rpt // CHUNK, zbody, 0)
    plsc.subcore_barrier()

    def body(j, carry):
        pltpu.async_copy(ones_v, acc.at[dst_v.at[j]], sem, add=True)
        return carry
    lax.fori_loop(0, NCH, body, 0)

    def drain(j, carry):
        pltpu.make_async_copy(ones_v, acc.at[dst_v.at[j]], sem).wait()
        return carry
    lax.fori_loop(0, NCH, drain, 0)
    plsc.subcore_barrier()

    @pl.when(c == 0)
    def _():
        pltpu.sync_copy(acc.at[pl.ds(s * rpt, rpt)], d0_hbm.at[pl.ds(s * rpt, rpt)])

    @pl.when(c == 1)
    def _():
        pltpu.sync_copy(acc.at[pl.ds(s * rpt, rpt)], d1_hbm.at[pl.ds(s * rpt, rpt)])


@functools.partial(
    pl.kernel,
    out_type=(jax.ShapeDtypeStruct((ACC_N, F), jnp.float32),
              jax.ShapeDtypeStruct((ACC_N, F), jnp.float32)),
    mesh=_mesh,
    scratch_types=(
        pltpu.VMEM((3, CHUNK), jnp.int32),
        pltpu.VMEM((3, CHUNK), jnp.int32),
        pltpu.VMEM((CHUNK, F), jnp.float32),
        pltpu.VMEM((CHUNK, F), jnp.float32),
        pltpu.VMEM((CHUNK, F), jnp.float32),
        pltpu.VMEM_SHARED((ACC_N, F), jnp.float32),
        pltpu.SemaphoreType.DMA,
        pltpu.SemaphoreType.DMA,
        pltpu.SemaphoreType.DMA,
        pltpu.SemaphoreType.DMA,
        pltpu.SemaphoreType.DMA,
        pltpu.SemaphoreType.DMA,
        pltpu.SemaphoreType.DMA,
        pltpu.SemaphoreType.DMA,
        pltpu.SemaphoreType.DMA,
        pltpu.SemaphoreType.DMA,
        pltpu.SemaphoreType.DMA,
        pltpu.SemaphoreType.DMA,
    ),
)
def _agg_kernel(y_hbm, src_hbm, dst_hbm, p0_hbm, p1_hbm,
                sidx, didx, rb0, rb1, rb2, acc,
                sg0, sg1, sg2, ss0, ss1, ss2,
                si0, si1, si2, sd0, sd1, sd2):
    c = lax.axis_index("c")
    s = lax.axis_index("s")
    rows = (rb0, rb1, rb2)
    sg = (sg0, sg1, sg2)
    ss = (ss0, ss1, ss2)
    si = (si0, si1, si2)
    sd = (sd0, sd1, sd2)

    def issue_gather(slot, buf_i):
        pltpu.async_copy(y_hbm.at[sidx.at[slot]], rows[buf_i], sg[buf_i])

    def wait_gather(buf_i):
        pltpu.make_async_copy(y_hbm.at[sidx.at[0]], rows[buf_i], sg[buf_i]).wait()

    def issue_scatter(slot, buf_i):
        pltpu.async_copy(rows[buf_i], acc.at[didx.at[slot]], ss[buf_i], add=True)

    def wait_scatter(buf_i):
        pltpu.make_async_copy(rows[buf_i], acc.at[didx.at[0]], ss[buf_i]).wait()

    for r in range(16):
        for q in range(F // 16):
            rb2[r, pl.ds(q * 16, 16)] = jnp.zeros((16,), jnp.float32)

    def zbody(k, carry):
        pltpu.async_copy(rb2.at[pl.ds(0, 16)],
                         acc.at[pl.ds(s * RPT + k * 16, 16)], sg0)
        return carry
    lax.fori_loop(0, RPT // 16, zbody, 0)
    pltpu.async_copy(rb2.at[pl.ds(0, RPT % 16)],
                     acc.at[pl.ds(s * RPT + (RPT // 16) * 16, RPT % 16)], sg0)

    for j in range(3):
        pltpu.sync_copy(src_hbm.at[c, s, j], sidx.at[j])
        pltpu.sync_copy(dst_hbm.at[c, s, j], didx.at[j])

    def zdrain(k, carry):
        pltpu.make_async_copy(rb2.at[pl.ds(0, 16)],
                              acc.at[pl.ds(s * RPT + k * 16, 16)], sg0).wait()
        return carry
    lax.fori_loop(0, RPT // 16, zdrain, 0)
    pltpu.make_async_copy(rb2.at[pl.ds(0, RPT % 16)],
                          acc.at[pl.ds(s * RPT + (RPT // 16) * 16, RPT % 16)],
                          sg0).wait()
    plsc.subcore_barrier()

    issue_gather(0, 0)
    issue_gather(1, 1)

    def body(t, carry):
        for u in range(3):
            j = 3 * t + u
            u1 = (u + 1) % 3
            u2 = (u + 2) % 3

            @pl.when(j >= 1)
            def _():
                wait_scatter(u2)

            @pl.when(jnp.logical_and(j >= 1, j + 2 < NCH))
            def _():
                pltpu.async_copy(dst_hbm.at[c, s, j + 2], didx.at[u2], sd[u2])

            @pl.when(jnp.logical_and(j >= 1, j + 2 < NCH))
            def _():
                pltpu.make_async_copy(
                    src_hbm.at[c, s, 0], sidx.at[u2], si[u2]).wait()

            @pl.when(j + 2 < NCH)
            def _():
                issue_gather(u2, u2)

            wait_gather(u)

            @pl.when(j + 3 < NCH)
            def _():
                pltpu.async_copy(src_hbm.at[c, s, j + 3], sidx.at[u], si[u])

            @pl.when(j >= 3)
            def _():
                pltpu.make_async_copy(
                    dst_hbm.at[c, s, 0], didx.at[u], sd[u]).wait()

            issue_scatter(u, u)
        return carry
    lax.fori_loop(0, NCH // 3, body, 0)
    wait_scatter(2)
    wait_gather(0)
    pltpu.make_async_copy(dst_hbm.at[c, s, 0], didx.at[0], sd[0]).wait()
    issue_scatter(0, 0)
    wait_scatter(0)
    plsc.subcore_barrier()

    @pl.when(c == 0)
    def _():
        pltpu.sync_copy(acc.at[pl.ds(s * RPT, RPT)], p0_hbm.at[pl.ds(s * RPT, RPT)])

    @pl.when(c == 1)
    def _():
        pltpu.sync_copy(acc.at[pl.ds(s * RPT, RPT)], p1_hbm.at[pl.ds(s * RPT, RPT)])


def _prep_body(x_ref, d0_ref, d1_ref, y_ref, dinv_ref):
    deg = d0_ref[...] + d1_ref[...]
    dinv = jnp.where(deg > 0, lax.rsqrt(jnp.maximum(deg, 1.0)), 0.0)
    dinv_ref[...] = dinv
    y_ref[...] = x_ref[...] * dinv


def _prep(x, d0r, d1r):
    return pl.pallas_call(
        _prep_body,
        grid=(G,),
        in_specs=[
            pl.BlockSpec((R, F), lambda i: (i, 0)),
            pl.BlockSpec((R, 1), lambda i: (i, 0)),
            pl.BlockSpec((R, 1), lambda i: (i, 0)),
        ],
        out_specs=[
            pl.BlockSpec((R, F), lambda i: (i, 0)),
            pl.BlockSpec((R, 1), lambda i: (i, 0)),
        ],
        out_shape=[
            jax.ShapeDtypeStruct((N, F), jnp.float32),
            jax.ShapeDtypeStruct((N, 1), jnp.float32),
        ],
    )(x, d0r, d1r)


def _combine_body(h_ref, p0_ref, p1_ref, dinv_ref, w_ref, b_ref,
                  out_ref, y_ref, *, relu, emit_y):
    dinv = dinv_ref[...]
    agg = (p0_ref[...] + p1_ref[...]) * dinv
    acc = jnp.dot(h_ref[...], w_ref[0], preferred_element_type=jnp.float32)
    acc = acc - jnp.dot(agg, w_ref[1], preferred_element_type=jnp.float32)
    acc = acc + b_ref[...]
    if relu:
        acc = jnp.maximum(acc, 0.0)
    out_ref[...] = acc
    if emit_y:
        y_ref[...] = acc * dinv


def _combine(h, p0, p1, dinv, w, b, relu, emit_y):
    body = functools.partial(_combine_body, relu=relu, emit_y=emit_y)
    n_out = 2 if emit_y else 1
    out = pl.pallas_call(
        body if emit_y else (lambda *a: body(*a, None)),
        grid=(G,),
        in_specs=[
            pl.BlockSpec((R, F), lambda i: (i, 0)),
            pl.BlockSpec((R, F), lambda i: (i, 0)),
            pl.BlockSpec((R, F), lambda i: (i, 0)),
            pl.BlockSpec((R, 1), lambda i: (i, 0)),
            pl.BlockSpec((2, F, F), lambda i: (0, 0, 0)),
            pl.BlockSpec((1, F), lambda i: (0, 0)),
        ],
        out_specs=[pl.BlockSpec((R, F), lambda i: (i, 0))] * n_out,
        out_shape=[jax.ShapeDtypeStruct((N, F), jnp.float32)] * n_out,
    )(h, p0, p1, dinv, w, b)
    return out if emit_y else out[0]


def kernel(x, edge_index, W1, b1, W2, b2):
    src = edge_index[0].astype(jnp.int32)
    dst = edge_index[1].astype(jnp.int32)
    npad = E_PAD - src.shape[0]
    ar = jnp.arange(npad, dtype=jnp.int32)
    pad_src = (ar * 131) % N
    pad_dst = N + (ar % (ACC_N - N))
    src_all = jnp.concatenate([src, pad_src]).reshape(NC, NS, NCH, CHUNK)
    dst_all = jnp.concatenate([dst, pad_dst]).reshape(NC, NS, NCH, CHUNK)

    d0, d1 = _deg_kernel(dst_all)
    d0r = d0[:N].reshape(N, 1)
    d1r = d1[:N].reshape(N, 1)
    y, dinv = _prep(x, d0r, d1r)

    b1r = b1.reshape(1, F)
    b2r = b2.reshape(1, F)
    h = x
    for _ in range(2):
        p0, p1 = _agg_kernel(y, src_all, dst_all)
        h, y = _combine(h, p0, p1, dinv, W1, b1r, relu=True, emit_y=True)
    p0, p1 = _agg_kernel(y, src_all, dst_all)
    return _combine(h, p0, p1, dinv, W2, b2r, relu=False, emit_y=False)


# speedup vs baseline: 26.6923x; 1.0244x over previous
import functools

import jax
import jax.numpy as jnp
from jax import lax
from jax.experimental import pallas as pl
from jax.experimental.pallas import tpu as pltpu
from jax.experimental.pallas import tpu_sc as plsc

N = 10000
F = 128
NC, NS = 2, 16
NW = NC * NS
CHUNK = 128
NCH = 79
EPT = NCH * CHUNK
E_PAD = NW * EPT
ACC_N = 10112
RPT = ACC_N // NS
DEG_N = 10240
R = 2000
G = N // R

_mesh = plsc.VectorSubcoreMesh(core_axis_name="c", subcore_axis_name="s")


@functools.partial(
    pl.kernel,
    out_type=(jax.ShapeDtypeStruct((DEG_N,), jnp.float32),
              jax.ShapeDtypeStruct((DEG_N,), jnp.float32)),
    mesh=_mesh,
    scratch_types=(
        pltpu.VMEM((NCH, CHUNK), jnp.int32),
        pltpu.VMEM((CHUNK,), jnp.float32),
        pltpu.VMEM((CHUNK,), jnp.float32),
        pltpu.VMEM_SHARED((DEG_N,), jnp.float32),
        pltpu.SemaphoreType.DMA,
    ),
)
def _deg_kernel(dst_hbm, d0_hbm, d1_hbm, dst_v, ones_v, zrow_v, acc, sem):
    c = lax.axis_index("c")
    s = lax.axis_index("s")
    for i in range(CHUNK // 16):
        ones_v[pl.ds(i * 16, 16)] = jnp.ones((16,), jnp.float32)
        zrow_v[pl.ds(i * 16, 16)] = jnp.zeros((16,), jnp.float32)
    pltpu.sync_copy(dst_hbm.at[c, s], dst_v)

    rpt = DEG_N // NS
    def zbody(k, carry):
        pltpu.sync_copy(zrow_v, acc.at[pl.ds(s * rpt + k * CHUNK, CHUNK)])
        return carry
    lax.fori_loop(0, rpt // CHUNK, zbody, 0)
    plsc.subcore_barrier()

    def body(j, carry):
        pltpu.async_copy(ones_v, acc.at[dst_v.at[j]], sem, add=True)
        return carry
    lax.fori_loop(0, NCH, body, 0)

    def drain(j, carry):
        pltpu.make_async_copy(ones_v, acc.at[dst_v.at[j]], sem).wait()
        return carry
    lax.fori_loop(0, NCH, drain, 0)
    plsc.subcore_barrier()

    @pl.when(c == 0)
    def _():
        pltpu.sync_copy(acc.at[pl.ds(s * rpt, rpt)], d0_hbm.at[pl.ds(s * rpt, rpt)])

    @pl.when(c == 1)
    def _():
        pltpu.sync_copy(acc.at[pl.ds(s * rpt, rpt)], d1_hbm.at[pl.ds(s * rpt, rpt)])


@functools.partial(
    pl.kernel,
    out_type=(jax.ShapeDtypeStruct((ACC_N, F), jnp.float32),
              jax.ShapeDtypeStruct((ACC_N, F), jnp.float32)),
    mesh=_mesh,
    scratch_types=(
        pltpu.VMEM((3, CHUNK), jnp.int32),
        pltpu.VMEM((3, CHUNK), jnp.int32),
        pltpu.VMEM((CHUNK, F), jnp.float32),
        pltpu.VMEM((CHUNK, F), jnp.float32),
        pltpu.VMEM((CHUNK, F), jnp.float32),
        pltpu.VMEM_SHARED((ACC_N, F), jnp.float32),
        pltpu.SemaphoreType.DMA,
        pltpu.SemaphoreType.DMA,
        pltpu.SemaphoreType.DMA,
        pltpu.SemaphoreType.DMA,
        pltpu.SemaphoreType.DMA,
        pltpu.SemaphoreType.DMA,
        pltpu.SemaphoreType.DMA,
        pltpu.SemaphoreType.DMA,
        pltpu.SemaphoreType.DMA,
        pltpu.SemaphoreType.DMA,
        pltpu.SemaphoreType.DMA,
        pltpu.SemaphoreType.DMA,
    ),
)
def _agg_kernel(y_hbm, src_hbm, dst_hbm, p0_hbm, p1_hbm,
                sidx, didx, rb0, rb1, rb2, acc,
                sg0, sg1, sg2, ss0, ss1, ss2,
                si0, si1, si2, sd0, sd1, sd2):
    c = lax.axis_index("c")
    s = lax.axis_index("s")
    rows = (rb0, rb1, rb2)
    sg = (sg0, sg1, sg2)
    ss = (ss0, ss1, ss2)
    si = (si0, si1, si2)
    sd = (sd0, sd1, sd2)

    def issue_gather(slot, buf_i):
        pltpu.async_copy(y_hbm.at[sidx.at[slot]], rows[buf_i], sg[buf_i])

    def wait_gather(buf_i):
        pltpu.make_async_copy(y_hbm.at[sidx.at[0]], rows[buf_i], sg[buf_i]).wait()

    def issue_scatter(slot, buf_i):
        pltpu.async_copy(rows[buf_i], acc.at[didx.at[slot]], ss[buf_i], add=True)

    def wait_scatter(buf_i):
        pltpu.make_async_copy(rows[buf_i], acc.at[didx.at[0]], ss[buf_i]).wait()

    for r in range(16):
        for q in range(F // 16):
            rb2[r, pl.ds(q * 16, 16)] = jnp.zeros((16,), jnp.float32)

    def zbody(k, carry):
        pltpu.async_copy(rb2.at[pl.ds(0, 16)],
                         acc.at[pl.ds(s * RPT + k * 16, 16)], sg0)
        return carry
    lax.fori_loop(0, RPT // 16, zbody, 0)
    pltpu.async_copy(rb2.at[pl.ds(0, RPT % 16)],
                     acc.at[pl.ds(s * RPT + (RPT // 16) * 16, RPT % 16)], sg0)

    for j in range(3):
        pltpu.sync_copy(src_hbm.at[c, s, j], sidx.at[j])
        pltpu.sync_copy(dst_hbm.at[c, s, j], didx.at[j])

    def zdrain(k, carry):
        pltpu.make_async_copy(rb2.at[pl.ds(0, 16)],
                              acc.at[pl.ds(s * RPT + k * 16, 16)], sg0).wait()
        return carry
    lax.fori_loop(0, RPT // 16, zdrain, 0)
    pltpu.make_async_copy(rb2.at[pl.ds(0, RPT % 16)],
                          acc.at[pl.ds(s * RPT + (RPT // 16) * 16, RPT % 16)],
                          sg0).wait()
    plsc.subcore_barrier()

    issue_gather(0, 0)
    issue_gather(1, 1)

    def body(t, carry):
        for u in range(3):
            j = 3 * t + u
            u1 = (u + 1) % 3
            u2 = (u + 2) % 3

            @pl.when(j >= 1)
            def _():
                wait_scatter(u2)

            @pl.when(jnp.logical_and(j >= 1, j + 2 < NCH))
            def _():
                pltpu.async_copy(dst_hbm.at[c, s, j + 2], didx.at[u2], sd[u2])

            @pl.when(jnp.logical_and(j >= 1, j + 2 < NCH))
            def _():
                pltpu.make_async_copy(
                    src_hbm.at[c, s, 0], sidx.at[u2], si[u2]).wait()

            @pl.when(j + 2 < NCH)
            def _():
                issue_gather(u2, u2)

            wait_gather(u)

            @pl.when(j + 3 < NCH)
            def _():
                pltpu.async_copy(src_hbm.at[c, s, j + 3], sidx.at[u], si[u])

            @pl.when(j >= 3)
            def _():
                pltpu.make_async_copy(
                    dst_hbm.at[c, s, 0], didx.at[u], sd[u]).wait()

            issue_scatter(u, u)
        return carry
    lax.fori_loop(0, NCH // 3, body, 0)
    wait_scatter(2)
    wait_gather(0)
    pltpu.make_async_copy(dst_hbm.at[c, s, 0], didx.at[0], sd[0]).wait()
    issue_scatter(0, 0)
    wait_scatter(0)
    plsc.subcore_barrier()

    @pl.when(c == 0)
    def _():
        pltpu.sync_copy(acc.at[pl.ds(s * RPT, RPT)], p0_hbm.at[pl.ds(s * RPT, RPT)])

    @pl.when(c == 1)
    def _():
        pltpu.sync_copy(acc.at[pl.ds(s * RPT, RPT)], p1_hbm.at[pl.ds(s * RPT, RPT)])


def _prep_body(x_ref, d0_ref, d1_ref, y_ref, dinv_ref):
    deg = d0_ref[...] + d1_ref[...]
    dinv = jnp.where(deg > 0, lax.rsqrt(jnp.maximum(deg, 1.0)), 0.0)
    dinv_ref[...] = dinv
    y_ref[...] = x_ref[...] * dinv


def _prep(x, d0r, d1r):
    return pl.pallas_call(
        _prep_body,
        grid=(G,),
        in_specs=[
            pl.BlockSpec((R, F), lambda i: (i, 0)),
            pl.BlockSpec((R, 1), lambda i: (i, 0)),
            pl.BlockSpec((R, 1), lambda i: (i, 0)),
        ],
        out_specs=[
            pl.BlockSpec((R, F), lambda i: (i, 0)),
            pl.BlockSpec((R, 1), lambda i: (i, 0)),
        ],
        out_shape=[
            jax.ShapeDtypeStruct((N, F), jnp.float32),
            jax.ShapeDtypeStruct((N, 1), jnp.float32),
        ],
    )(x, d0r, d1r)


def _combine_body(h_ref, p0_ref, p1_ref, dinv_ref, w_ref, b_ref,
                  out_ref, y_ref, *, relu, emit_y):
    dinv = dinv_ref[...]
    agg = (p0_ref[...] + p1_ref[...]) * dinv
    acc = jnp.dot(h_ref[...], w_ref[0], preferred_element_type=jnp.float32)
    acc = acc - jnp.dot(agg, w_ref[1], preferred_element_type=jnp.float32)
    acc = acc + b_ref[...]
    if relu:
        acc = jnp.maximum(acc, 0.0)
    out_ref[...] = acc
    if emit_y:
        y_ref[...] = acc * dinv


def _combine(h, p0, p1, dinv, w, b, relu, emit_y):
    body = functools.partial(_combine_body, relu=relu, emit_y=emit_y)
    n_out = 2 if emit_y else 1
    out = pl.pallas_call(
        body if emit_y else (lambda *a: body(*a, None)),
        grid=(G,),
        in_specs=[
            pl.BlockSpec((R, F), lambda i: (i, 0)),
            pl.BlockSpec((R, F), lambda i: (i, 0)),
            pl.BlockSpec((R, F), lambda i: (i, 0)),
            pl.BlockSpec((R, 1), lambda i: (i, 0)),
            pl.BlockSpec((2, F, F), lambda i: (0, 0, 0)),
            pl.BlockSpec((1, F), lambda i: (0, 0)),
        ],
        out_specs=[pl.BlockSpec((R, F), lambda i: (i, 0))] * n_out,
        out_shape=[jax.ShapeDtypeStruct((N, F), jnp.float32)] * n_out,
    )(h, p0, p1, dinv, w, b)
    return out if emit_y else out[0]


def kernel(x, edge_index, W1, b1, W2, b2):
    src = edge_index[0].astype(jnp.int32)
    dst = edge_index[1].astype(jnp.int32)
    npad = E_PAD - src.shape[0]
    ar = jnp.arange(npad, dtype=jnp.int32)
    pad_src = (ar * 131) % N
    pad_dst = N + (ar % (ACC_N - N))
    src_all = jnp.concatenate([src, pad_src]).reshape(NC, NS, NCH, CHUNK)
    dst_all = jnp.concatenate([dst, pad_dst]).reshape(NC, NS, NCH, CHUNK)

    d0, d1 = _deg_kernel(dst_all)
    d0r = d0[:N].reshape(N, 1)
    d1r = d1[:N].reshape(N, 1)
    y, dinv = _prep(x, d0r, d1r)

    b1r = b1.reshape(1, F)
    b2r = b2.reshape(1, F)
    h = x
    for _ in range(2):
        p0, p1 = _agg_kernel(y, src_all, dst_all)
        h, y = _combine(h, p0, p1, dinv, W1, b1r, relu=True, emit_y=True)
    p0, p1 = _agg_kernel(y, src_all, dst_all)
    return _combine(h, p0, p1, dinv, W2, b2r, relu=False, emit_y=False)


# speedup vs baseline: 27.0811x; 1.0146x over previous
import functools

import jax
import jax.numpy as jnp
from jax import lax
from jax.experimental import pallas as pl
from jax.experimental.pallas import tpu as pltpu
from jax.experimental.pallas import tpu_sc as plsc

N = 10000
F = 128
NC, NS = 2, 16
NW = NC * NS
CHUNK = 128
NCH = 79
EPT = NCH * CHUNK
E_PAD = NW * EPT
ACC_N = 10112
RPT = ACC_N // NS
DEG_N = 10240
R = 5000
G = N // R

_mesh = plsc.VectorSubcoreMesh(core_axis_name="c", subcore_axis_name="s")


@functools.partial(
    pl.kernel,
    out_type=(jax.ShapeDtypeStruct((DEG_N,), jnp.float32),
              jax.ShapeDtypeStruct((DEG_N,), jnp.float32)),
    mesh=_mesh,
    scratch_types=(
        pltpu.VMEM((NCH, CHUNK), jnp.int32),
        pltpu.VMEM((CHUNK,), jnp.float32),
        pltpu.VMEM((CHUNK,), jnp.float32),
        pltpu.VMEM_SHARED((DEG_N,), jnp.float32),
        pltpu.SemaphoreType.DMA,
    ),
)
def _deg_kernel(dst_hbm, d0_hbm, d1_hbm, dst_v, ones_v, zrow_v, acc, sem):
    c = lax.axis_index("c")
    s = lax.axis_index("s")
    for i in range(CHUNK // 16):
        ones_v[pl.ds(i * 16, 16)] = jnp.ones((16,), jnp.float32)
        zrow_v[pl.ds(i * 16, 16)] = jnp.zeros((16,), jnp.float32)
    pltpu.sync_copy(dst_hbm.at[c, s], dst_v)

    rpt = DEG_N // NS
    def zbody(k, carry):
        pltpu.sync_copy(zrow_v, acc.at[pl.ds(s * rpt + k * CHUNK, CHUNK)])
        return carry
    lax.fori_loop(0, rpt // CHUNK, zbody, 0)
    plsc.subcore_barrier()

    def body(j, carry):
        pltpu.async_copy(ones_v, acc.at[dst_v.at[j]], sem, add=True)
        return carry
    lax.fori_loop(0, NCH, body, 0)

    def drain(j, carry):
        pltpu.make_async_copy(ones_v, acc.at[dst_v.at[j]], sem).wait()
        return carry
    lax.fori_loop(0, NCH, drain, 0)
    plsc.subcore_barrier()

    @pl.when(c == 0)
    def _():
        pltpu.sync_copy(acc.at[pl.ds(s * rpt, rpt)], d0_hbm.at[pl.ds(s * rpt, rpt)])

    @pl.when(c == 1)
    def _():
        pltpu.sync_copy(acc.at[pl.ds(s * rpt, rpt)], d1_hbm.at[pl.ds(s * rpt, rpt)])


@functools.partial(
    pl.kernel,
    out_type=(jax.ShapeDtypeStruct((ACC_N, F), jnp.float32),
              jax.ShapeDtypeStruct((ACC_N, F), jnp.float32)),
    mesh=_mesh,
    scratch_types=(
        pltpu.VMEM((3, CHUNK), jnp.int32),
        pltpu.VMEM((3, CHUNK), jnp.int32),
        pltpu.VMEM((CHUNK, F), jnp.float32),
        pltpu.VMEM((CHUNK, F), jnp.float32),
        pltpu.VMEM((CHUNK, F), jnp.float32),
        pltpu.VMEM_SHARED((ACC_N, F), jnp.float32),
        pltpu.SemaphoreType.DMA,
        pltpu.SemaphoreType.DMA,
        pltpu.SemaphoreType.DMA,
        pltpu.SemaphoreType.DMA,
        pltpu.SemaphoreType.DMA,
        pltpu.SemaphoreType.DMA,
        pltpu.SemaphoreType.DMA,
        pltpu.SemaphoreType.DMA,
        pltpu.SemaphoreType.DMA,
        pltpu.SemaphoreType.DMA,
        pltpu.SemaphoreType.DMA,
        pltpu.SemaphoreType.DMA,
    ),
)
def _agg_kernel(y_hbm, src_hbm, dst_hbm, p0_hbm, p1_hbm,
                sidx, didx, rb0, rb1, rb2, acc,
                sg0, sg1, sg2, ss0, ss1, ss2,
                si0, si1, si2, sd0, sd1, sd2):
    c = lax.axis_index("c")
    s = lax.axis_index("s")
    rows = (rb0, rb1, rb2)
    sg = (sg0, sg1, sg2)
    ss = (ss0, ss1, ss2)
    si = (si0, si1, si2)
    sd = (sd0, sd1, sd2)

    def issue_gather(slot, buf_i):
        pltpu.async_copy(y_hbm.at[sidx.at[slot]], rows[buf_i], sg[buf_i])

    def wait_gather(buf_i):
        pltpu.make_async_copy(y_hbm.at[sidx.at[0]], rows[buf_i], sg[buf_i]).wait()

    def issue_scatter(slot, buf_i):
        pltpu.async_copy(rows[buf_i], acc.at[didx.at[slot]], ss[buf_i], add=True)

    def wait_scatter(buf_i):
        pltpu.make_async_copy(rows[buf_i], acc.at[didx.at[0]], ss[buf_i]).wait()

    for r in range(16):
        for q in range(F // 16):
            rb2[r, pl.ds(q * 16, 16)] = jnp.zeros((16,), jnp.float32)

    def zbody(k, carry):
        pltpu.async_copy(rb2.at[pl.ds(0, 16)],
                         acc.at[pl.ds(s * RPT + k * 16, 16)], sg0)
        return carry
    lax.fori_loop(0, RPT // 16, zbody, 0)
    pltpu.async_copy(rb2.at[pl.ds(0, RPT % 16)],
                     acc.at[pl.ds(s * RPT + (RPT // 16) * 16, RPT % 16)], sg0)

    for j in range(3):
        pltpu.sync_copy(src_hbm.at[c, s, j], sidx.at[j])
        pltpu.sync_copy(dst_hbm.at[c, s, j], didx.at[j])

    def zdrain(k, carry):
        pltpu.make_async_copy(rb2.at[pl.ds(0, 16)],
                              acc.at[pl.ds(s * RPT + k * 16, 16)], sg0).wait()
        return carry
    lax.fori_loop(0, RPT // 16, zdrain, 0)
    pltpu.make_async_copy(rb2.at[pl.ds(0, RPT % 16)],
                          acc.at[pl.ds(s * RPT + (RPT // 16) * 16, RPT % 16)],
                          sg0).wait()
    plsc.subcore_barrier()

    issue_gather(0, 0)
    issue_gather(1, 1)

    def body(t, carry):
        for u in range(3):
            j = 3 * t + u
            u1 = (u + 1) % 3
            u2 = (u + 2) % 3

            @pl.when(j >= 1)
            def _():
                wait_scatter(u2)

            @pl.when(jnp.logical_and(j >= 1, j + 2 < NCH))
            def _():
                pltpu.async_copy(dst_hbm.at[c, s, j + 2], didx.at[u2], sd[u2])

            @pl.when(jnp.logical_and(j >= 1, j + 2 < NCH))
            def _():
                pltpu.make_async_copy(
                    src_hbm.at[c, s, 0], sidx.at[u2], si[u2]).wait()

            @pl.when(j + 2 < NCH)
            def _():
                issue_gather(u2, u2)

            wait_gather(u)

            @pl.when(j + 3 < NCH)
            def _():
                pltpu.async_copy(src_hbm.at[c, s, j + 3], sidx.at[u], si[u])

            @pl.when(j >= 3)
            def _():
                pltpu.make_async_copy(
                    dst_hbm.at[c, s, 0], didx.at[u], sd[u]).wait()

            issue_scatter(u, u)
        return carry
    lax.fori_loop(0, NCH // 3, body, 0)
    wait_scatter(2)
    wait_gather(0)
    pltpu.make_async_copy(dst_hbm.at[c, s, 0], didx.at[0], sd[0]).wait()
    issue_scatter(0, 0)
    wait_scatter(0)
    plsc.subcore_barrier()

    @pl.when(c == 0)
    def _():
        pltpu.sync_copy(acc.at[pl.ds(s * RPT, RPT)], p0_hbm.at[pl.ds(s * RPT, RPT)])

    @pl.when(c == 1)
    def _():
        pltpu.sync_copy(acc.at[pl.ds(s * RPT, RPT)], p1_hbm.at[pl.ds(s * RPT, RPT)])


def _prep_body(x_ref, d0_ref, d1_ref, y_ref, dinv_ref):
    deg = d0_ref[...] + d1_ref[...]
    dinv = jnp.where(deg > 0, lax.rsqrt(jnp.maximum(deg, 1.0)), 0.0)
    dinv_ref[...] = dinv
    y_ref[...] = x_ref[...] * dinv


def _prep(x, d0r, d1r):
    return pl.pallas_call(
        _prep_body,
        grid=(G,),
        in_specs=[
            pl.BlockSpec((R, F), lambda i: (i, 0)),
            pl.BlockSpec((R, 1), lambda i: (i, 0)),
            pl.BlockSpec((R, 1), lambda i: (i, 0)),
        ],
        out_specs=[
            pl.BlockSpec((R, F), lambda i: (i, 0)),
            pl.BlockSpec((R, 1), lambda i: (i, 0)),
        ],
        out_shape=[
            jax.ShapeDtypeStruct((N, F), jnp.float32),
            jax.ShapeDtypeStruct((N, 1), jnp.float32),
        ],
    )(x, d0r, d1r)


def _combine_body(h_ref, p0_ref, p1_ref, dinv_ref, w_ref, b_ref,
                  out_ref, y_ref, *, relu, emit_y):
    dinv = dinv_ref[...]
    agg = (p0_ref[...] + p1_ref[...]) * dinv
    acc = jnp.dot(h_ref[...], w_ref[0], preferred_element_type=jnp.float32)
    acc = acc - jnp.dot(agg, w_ref[1], preferred_element_type=jnp.float32)
    acc = acc + b_ref[...]
    if relu:
        acc = jnp.maximum(acc, 0.0)
    out_ref[...] = acc
    if emit_y:
        y_ref[...] = acc * dinv


def _combine(h, p0, p1, dinv, w, b, relu, emit_y):
    body = functools.partial(_combine_body, relu=relu, emit_y=emit_y)
    n_out = 2 if emit_y else 1
    out = pl.pallas_call(
        body if emit_y else (lambda *a: body(*a, None)),
        grid=(G,),
        in_specs=[
            pl.BlockSpec((R, F), lambda i: (i, 0)),
            pl.BlockSpec((R, F), lambda i: (i, 0)),
            pl.BlockSpec((R, F), lambda i: (i, 0)),
            pl.BlockSpec((R, 1), lambda i: (i, 0)),
            pl.BlockSpec((2, F, F), lambda i: (0, 0, 0)),
            pl.BlockSpec((1, F), lambda i: (0, 0)),
        ],
        out_specs=[pl.BlockSpec((R, F), lambda i: (i, 0))] * n_out,
        out_shape=[jax.ShapeDtypeStruct((N, F), jnp.float32)] * n_out,
    )(h, p0, p1, dinv, w, b)
    return out if emit_y else out[0]


def kernel(x, edge_index, W1, b1, W2, b2):
    src = edge_index[0].astype(jnp.int32)
    dst = edge_index[1].astype(jnp.int32)
    npad = E_PAD - src.shape[0]
    ar = jnp.arange(npad, dtype=jnp.int32)
    pad_src = (ar * 131) % N
    pad_dst = N + (ar % (ACC_N - N))
    src_all = jnp.concatenate([src, pad_src]).reshape(NC, NS, NCH, CHUNK)
    dst_all = jnp.concatenate([dst, pad_dst]).reshape(NC, NS, NCH, CHUNK)

    d0, d1 = _deg_kernel(dst_all)
    d0r = d0[:N].reshape(N, 1)
    d1r = d1[:N].reshape(N, 1)
    y, dinv = _prep(x, d0r, d1r)

    b1r = b1.reshape(1, F)
    b2r = b2.reshape(1, F)
    h = x
    for _ in range(2):
        p0, p1 = _agg_kernel(y, src_all, dst_all)
        h, y = _combine(h, p0, p1, dinv, W1, b1r, relu=True, emit_y=True)
    p0, p1 = _agg_kernel(y, src_all, dst_all)
    return _combine(h, p0, p1, dinv, W2, b2r, relu=False, emit_y=False)
